# baseline (speedup 1.0000x reference)
import functools

import jax
import jax.numpy as jnp
import numpy as np
from jax import lax
from jax.experimental import pallas as pl
from jax.experimental.pallas import tpu as pltpu
from jax.experimental.pallas import tpu_sc as plsc


def _gauss_coeffs(k_size=5, sigma=1.0):
    center = k_size // 2
    x, y = np.mgrid[0 - center:k_size - center, 0 - center:k_size - center]
    g = 1.0 / (2 * np.pi * sigma) * np.exp(-(np.square(x) + np.square(y)) / (2 * np.square(sigma)))
    return g.astype(np.float32)


def _conf_body(psm_ref, conf_ref, *, gx, gy, A, H, W):
    conf = jax.nn.sigmoid(psm_ref[0, 0])
    for a in range(1, A):
        conf = jnp.maximum(conf, jax.nn.sigmoid(psm_ref[0, a]))
    kw = gx.shape[0]
    kh = gy.shape[0]
    ph, pw = (kh - 1) // 2, (kw - 1) // 2
    zc = jnp.zeros((H, pw), jnp.float32)
    p = jnp.concatenate([zc, conf, zc], axis=1)
    row = jnp.zeros((H, W), jnp.float32)
    for dx in range(kw):
        row = row + float(gx[dx]) * p[:, dx:dx + W]
    zr = jnp.zeros((ph, W), jnp.float32)
    q = jnp.concatenate([zr, row, zr], axis=0)
    acc = jnp.zeros((H, W), jnp.float32)
    for dy in range(kh):
        acc = acc + float(gy[dy]) * q[dy:dy + H, :]
    conf_ref[0, 0] = acc


def _sc_thr_body(conf_hbm, out_hbm, buf_v, hist_v, sfx_v, *, n_slices, hw, K):
    info = plsc.get_sparse_core_info()
    nc = info.num_cores
    wid = lax.axis_index("s") * nc + lax.axis_index("c")

    @pl.when(wid < n_slices)
    def _():
        pltpu.sync_copy(conf_hbm.at[wid], buf_v)
        n_vregs = hw // 16
        ones16 = jnp.ones((16,), jnp.int32)
        iota16 = lax.iota(jnp.int32, 16)
        zeros16 = jnp.zeros((16,), jnp.int32)
        remaining = jnp.int32(K)
        pref = jnp.int32(0)
        for level in range(4):
            shift = 24 - 8 * level

            @plsc.parallel_loop(0, 256, 1, unroll=8)
            def zero_body(j):
                hist_v[pl.ds(j * 16, 16)] = zeros16

            prefp = lax.shift_right_logical(pref, shift + 8) if level else jnp.int32(0)

            @plsc.parallel_loop(0, n_vregs, 1, unroll=32)
            def scan_body(i, _shift=shift, _level=level, _prefp=prefp):
                v = buf_v[pl.ds(i * 16, 16)]
                byte = jnp.bitwise_and(lax.shift_right_logical(v, _shift), 0xFF)
                slot = lax.shift_left(byte, 4) + iota16
                if _level:
                    m = lax.shift_right_logical(v, _shift + 8) == _prefp
                    plsc.addupdate_scatter(hist_v, [slot], ones16, mask=m)
                else:
                    plsc.addupdate_scatter(hist_v, [slot], ones16)

            @plsc.parallel_loop(0, 256, 1, carry=zeros16)
            def sfx_body(i, vacc):
                b = 255 - i
                vacc = vacc + hist_v[pl.ds(b * 16, 16)]
                sfx_v[pl.ds(b * 16, 16)] = vacc
                return vacc

            lo = jnp.int32(0)
            hi = jnp.int32(255)
            for _ in range(8):
                mid = hi - lax.shift_right_logical(hi - lo, 1)
                s_mid = jnp.sum(sfx_v[pl.ds(mid * 16, 16)])
                big = s_mid >= remaining
                lo = jnp.where(big, mid, lo)
                hi = jnp.where(big, hi, mid - 1)
            best = lo
            nxt = jnp.minimum(best + 1, 255)
            s_nxt = jnp.sum(sfx_v[pl.ds(nxt * 16, 16)])
            count_above = jnp.where(best == 255, 0, s_nxt)
            remaining = remaining - count_above
            pref = jnp.bitwise_or(pref, lax.shift_left(best, shift))
        buf_v[pl.ds(0, 16)] = jnp.full((16,), pref, jnp.int32)
        pltpu.sync_copy(buf_v.at[pl.ds(0, 16)], out_hbm.at[pl.ds(wid * 16, 16)])


def _fusion_body(x_ref, c_ref, t_ref, o_ref, *, L, C):
    isc = float(1.0 / np.sqrt(C))
    x0 = x_ref[0, 0]
    s = [jnp.sum(x0 * x0, axis=0) * isc]
    mm = []
    for m in range(1, L):
        d = jnp.sum(x0 * x_ref[0, m], axis=0)
        mk = (c_ref[0, m] >= t_ref[0, 0, m]).astype(jnp.float32)
        mm.append(mk)
        s.append(mk * d * isc)
    smax = s[0]
    for m in range(1, L):
        smax = jnp.maximum(smax, s[m])
    e = [jnp.exp(sm - smax) for sm in s]
    den = e[0]
    for m in range(1, L):
        den = den + e[m]
    inv_den = 1.0 / den
    acc = (e[0] * inv_den)[None] * x0
    for m in range(1, L):
        w = e[m] * mm[m - 1] * inv_den
        acc = acc + w[None] * x_ref[0, m]
    o_ref[0] = acc


def kernel(x, psm_single, record_len, pairwise_t_matrix):
    N, C, H, W = x.shape
    B = record_len.shape[0]
    L = N // B
    A = psm_single.shape[1]
    K = (H * W) // 2
    HT = 32
    center = 5 // 2
    r = np.arange(5) - center
    gx = np.exp(-np.square(r) / 2.0).astype(np.float32)
    gy = (np.exp(-np.square(r) / 2.0) / (2 * np.pi)).astype(np.float32)

    conf = pl.pallas_call(
        functools.partial(_conf_body, gx=gx, gy=gy, A=A, H=H, W=W),
        grid=(N,),
        in_specs=[pl.BlockSpec((1, A, H, W), lambda i: (i, 0, 0, 0))],
        out_specs=pl.BlockSpec((1, 1, H, W), lambda i: (i // L, i % L, 0, 0)),
        out_shape=jax.ShapeDtypeStruct((B, L, H, W), jnp.float32),
    )(psm_single)

    sc_thr = functools.partial(
        pl.kernel,
        out_type=jax.ShapeDtypeStruct((N * 16,), jnp.int32),
        mesh=plsc.VectorSubcoreMesh(core_axis_name="c", subcore_axis_name="s"),
        compiler_params=pltpu.CompilerParams(needs_layout_passes=False),
        scratch_types=[
            pltpu.VMEM((H * W,), jnp.int32),
            pltpu.VMEM((4096,), jnp.int32),
            pltpu.VMEM((256,), jnp.int32),
        ],
    )(functools.partial(_sc_thr_body, n_slices=N, hw=H * W, K=K))
    conf_bits = lax.bitcast_convert_type(conf, jnp.int32).reshape(N, H * W)
    thr_rows = sc_thr(conf_bits)
    thr = lax.bitcast_convert_type(thr_rows[::16], jnp.float32).reshape(B, 1, L)

    xs = x.reshape(B, L, C, H, W)
    fused = pl.pallas_call(
        functools.partial(_fusion_body, L=L, C=C),
        grid=(B, H // HT),
        in_specs=[
            pl.BlockSpec((1, L, C, HT, W), lambda b, t: (b, 0, 0, t, 0)),
            pl.BlockSpec((1, L, HT, W), lambda b, t: (b, 0, t, 0)),
            pl.BlockSpec((1, 1, L), lambda b, t: (b, 0, 0)),
        ],
        out_specs=pl.BlockSpec((1, C, HT, W), lambda b, t: (b, 0, t, 0)),
        out_shape=jax.ShapeDtypeStruct((B, C, H, W), jnp.float32),
    )(xs, conf, thr)

    rate = jnp.float32(K / (H * W))
    return fused, rate


# speedup vs baseline: 1.0228x; 1.0228x over previous
import functools

import jax
import jax.numpy as jnp
import numpy as np
from jax import lax
from jax.experimental import pallas as pl
from jax.experimental.pallas import tpu as pltpu
from jax.experimental.pallas import tpu_sc as plsc


def _gauss_coeffs(k_size=5, sigma=1.0):
    center = k_size // 2
    x, y = np.mgrid[0 - center:k_size - center, 0 - center:k_size - center]
    g = 1.0 / (2 * np.pi * sigma) * np.exp(-(np.square(x) + np.square(y)) / (2 * np.square(sigma)))
    return g.astype(np.float32)


def _conf_body(psm_ref, conf_ref, *, gx, gy, A, H, W):
    conf = jax.nn.sigmoid(psm_ref[0, 0])
    for a in range(1, A):
        conf = jnp.maximum(conf, jax.nn.sigmoid(psm_ref[0, a]))
    kw = gx.shape[0]
    kh = gy.shape[0]
    ph, pw = (kh - 1) // 2, (kw - 1) // 2
    zc = jnp.zeros((H, pw), jnp.float32)
    p = jnp.concatenate([zc, conf, zc], axis=1)
    row = jnp.zeros((H, W), jnp.float32)
    for dx in range(kw):
        row = row + float(gx[dx]) * p[:, dx:dx + W]
    zr = jnp.zeros((ph, W), jnp.float32)
    q = jnp.concatenate([zr, row, zr], axis=0)
    acc = jnp.zeros((H, W), jnp.float32)
    for dy in range(kh):
        acc = acc + float(gy[dy]) * q[dy:dy + H, :]
    conf_ref[0, 0] = acc


def _sc_thr_body(conf_hbm, out_hbm, buf_v, hist_v, sfx_v, *, n_slices, hw, K):
    info = plsc.get_sparse_core_info()
    nc = info.num_cores
    wid = lax.axis_index("s") * nc + lax.axis_index("c")

    @pl.when(wid < n_slices)
    def _():
        pltpu.sync_copy(conf_hbm.at[wid], buf_v)
        n_vregs = hw // 16
        ones16 = jnp.ones((16,), jnp.int32)
        iota16 = lax.iota(jnp.int32, 16)
        zeros16 = jnp.zeros((16,), jnp.int32)
        remaining = jnp.int32(K)
        pref = jnp.int32(0)
        for level in range(4):
            shift = 24 - 8 * level

            @plsc.parallel_loop(0, 256, 1, unroll=8)
            def zero_body(j):
                hist_v[pl.ds(j * 16, 16)] = zeros16

            prefp = lax.shift_right_logical(pref, shift + 8) if level else jnp.int32(0)

            @plsc.parallel_loop(0, n_vregs, 1, unroll=16)
            def scan_body(i, _shift=shift, _level=level, _prefp=prefp):
                v = buf_v[pl.ds(i * 16, 16)]
                byte = jnp.bitwise_and(lax.shift_right_logical(v, _shift), 0xFF)
                slot = lax.shift_left(byte, 4) + iota16
                if _level:
                    m = lax.shift_right_logical(v, _shift + 8) == _prefp
                    plsc.addupdate_scatter(hist_v, [slot], ones16, mask=m)
                else:
                    plsc.addupdate_scatter(hist_v, [slot], ones16)

            @plsc.parallel_loop(0, 256, 1, carry=zeros16)
            def sfx_body(i, vacc):
                b = 255 - i
                vacc = vacc + hist_v[pl.ds(b * 16, 16)]
                sfx_v[pl.ds(b * 16, 16)] = vacc
                return vacc

            lo = jnp.int32(0)
            hi = jnp.int32(255)
            for _ in range(8):
                mid = hi - lax.shift_right_logical(hi - lo, 1)
                s_mid = jnp.sum(sfx_v[pl.ds(mid * 16, 16)])
                big = s_mid >= remaining
                lo = jnp.where(big, mid, lo)
                hi = jnp.where(big, hi, mid - 1)
            best = lo
            nxt = jnp.minimum(best + 1, 255)
            s_nxt = jnp.sum(sfx_v[pl.ds(nxt * 16, 16)])
            count_above = jnp.where(best == 255, 0, s_nxt)
            remaining = remaining - count_above
            pref = jnp.bitwise_or(pref, lax.shift_left(best, shift))
        buf_v[pl.ds(0, 16)] = jnp.full((16,), pref, jnp.int32)
        pltpu.sync_copy(buf_v.at[pl.ds(0, 16)], out_hbm.at[pl.ds(wid * 16, 16)])


def _fusion_body(x_ref, c_ref, t_ref, o_ref, *, L, C):
    isc = float(1.0 / np.sqrt(C))
    x0 = x_ref[0, 0]
    s = [jnp.sum(x0 * x0, axis=0) * isc]
    mm = []
    for m in range(1, L):
        d = jnp.sum(x0 * x_ref[0, m], axis=0)
        mk = (c_ref[0, m] >= t_ref[0, 0, m]).astype(jnp.float32)
        mm.append(mk)
        s.append(mk * d * isc)
    smax = s[0]
    for m in range(1, L):
        smax = jnp.maximum(smax, s[m])
    e = [jnp.exp(sm - smax) for sm in s]
    den = e[0]
    for m in range(1, L):
        den = den + e[m]
    inv_den = 1.0 / den
    acc = (e[0] * inv_den)[None] * x0
    for m in range(1, L):
        w = e[m] * mm[m - 1] * inv_den
        acc = acc + w[None] * x_ref[0, m]
    o_ref[0] = acc


def kernel(x, psm_single, record_len, pairwise_t_matrix):
    N, C, H, W = x.shape
    B = record_len.shape[0]
    L = N // B
    A = psm_single.shape[1]
    K = (H * W) // 2
    HT = 32
    center = 5 // 2
    r = np.arange(5) - center
    gx = np.exp(-np.square(r) / 2.0).astype(np.float32)
    gy = (np.exp(-np.square(r) / 2.0) / (2 * np.pi)).astype(np.float32)

    conf = pl.pallas_call(
        functools.partial(_conf_body, gx=gx, gy=gy, A=A, H=H, W=W),
        grid=(N,),
        in_specs=[pl.BlockSpec((1, A, H, W), lambda i: (i, 0, 0, 0))],
        out_specs=pl.BlockSpec((1, 1, H, W), lambda i: (i // L, i % L, 0, 0)),
        out_shape=jax.ShapeDtypeStruct((B, L, H, W), jnp.float32),
    )(psm_single)

    sc_thr = functools.partial(
        pl.kernel,
        out_type=jax.ShapeDtypeStruct((N * 16,), jnp.int32),
        mesh=plsc.VectorSubcoreMesh(core_axis_name="c", subcore_axis_name="s"),
        compiler_params=pltpu.CompilerParams(needs_layout_passes=False),
        scratch_types=[
            pltpu.VMEM((H * W,), jnp.int32),
            pltpu.VMEM((4096,), jnp.int32),
            pltpu.VMEM((256,), jnp.int32),
        ],
    )(functools.partial(_sc_thr_body, n_slices=N, hw=H * W, K=K))
    conf_bits = lax.bitcast_convert_type(conf, jnp.int32).reshape(N, H * W)
    thr_rows = sc_thr(conf_bits)
    thr = lax.bitcast_convert_type(thr_rows[::16], jnp.float32).reshape(B, 1, L)

    xs = x.reshape(B, L, C, H, W)
    fused = pl.pallas_call(
        functools.partial(_fusion_body, L=L, C=C),
        grid=(B, H // HT),
        in_specs=[
            pl.BlockSpec((1, L, C, HT, W), lambda b, t: (b, 0, 0, t, 0)),
            pl.BlockSpec((1, L, HT, W), lambda b, t: (b, 0, t, 0)),
            pl.BlockSpec((1, 1, L), lambda b, t: (b, 0, 0)),
        ],
        out_specs=pl.BlockSpec((1, C, HT, W), lambda b, t: (b, 0, t, 0)),
        out_shape=jax.ShapeDtypeStruct((B, C, H, W), jnp.float32),
    )(xs, conf, thr)

    rate = jnp.float32(K / (H * W))
    return fused, rate
